# Initial kernel scaffold; baseline (speedup 1.0000x reference)
import jax
import jax.numpy as jnp
from jax.experimental import pallas as pl


def kernel(x, idx_tensor):
    raise NotImplementedError("write your pallas kernel here")


# speedup vs baseline: 173.2525x; 173.2525x over previous
import functools

import jax
import jax.numpy as jnp
from jax import lax
from jax.experimental import pallas as pl
from jax.experimental.pallas import tpu as pltpu
from jax.experimental.pallas import tpu_sc as plsc

_SHAPE = (64, 8192)
_TABLE = _SHAPE[1]
_TOTAL = _SHAPE[0] * _SHAPE[1]

_info = plsc.get_sparse_core_info()
_NC, _NS, _L = _info.num_cores, _info.num_subcores, _info.num_lanes
_NW = _NC * _NS
_CHUNK = _TOTAL // _NW
_VECS = _CHUNK // _L


def _gather_body(x_hbm, idx_hbm, out_hbm, table_v, idx_v, out_v):
    wid = lax.axis_index("s") * _NC + lax.axis_index("c")
    base = wid * _CHUNK
    pltpu.sync_copy(x_hbm, table_v)
    pltpu.sync_copy(idx_hbm.at[pl.ds(base, _CHUNK)], idx_v)

    def step(i, carry):
        off = i * _L
        iv = idx_v[pl.ds(off, _L)]
        out_v[pl.ds(off, _L)] = plsc.load_gather(table_v, [iv])
        return carry

    lax.fori_loop(0, _VECS, step, 0, unroll=8)
    pltpu.sync_copy(out_v, out_hbm.at[pl.ds(base, _CHUNK)])


_gather = pl.kernel(
    _gather_body,
    out_type=jax.ShapeDtypeStruct((_TOTAL,), jnp.float32),
    mesh=plsc.VectorSubcoreMesh(core_axis_name="c", subcore_axis_name="s"),
    scratch_types=[
        pltpu.VMEM((_TABLE,), jnp.float32),
        pltpu.VMEM((_CHUNK,), jnp.int32),
        pltpu.VMEM((_CHUNK,), jnp.float32),
    ],
    compiler_params=pltpu.CompilerParams(needs_layout_passes=False),
)


def kernel(x, idx_tensor):
    idx_flat = idx_tensor.astype(jnp.int32).reshape(_TOTAL)
    return _gather(x, idx_flat).reshape(_SHAPE)


# speedup vs baseline: 211.1200x; 1.2186x over previous
import functools

import jax
import jax.numpy as jnp
from jax import lax
from jax.experimental import pallas as pl
from jax.experimental.pallas import tpu as pltpu
from jax.experimental.pallas import tpu_sc as plsc

_SHAPE = (64, 8192)
_TABLE = _SHAPE[1]
_TOTAL = _SHAPE[0] * _SHAPE[1]

_info = plsc.get_sparse_core_info()
_NC, _NS, _L = _info.num_cores, _info.num_subcores, _info.num_lanes
_NW = _NC * _NS
_CHUNK = _TOTAL // _NW
_VECS = _CHUNK // _L


_NSUB = 4
_SUBC = _CHUNK // _NSUB
_SUBV = _SUBC // _L


def _gather_body(x_hbm, idx_hbm, out_hbm, table_v, idx_v, out_v,
                 sem_t, sem_i, sem_o):
    wid = lax.axis_index("s") * _NC + lax.axis_index("c")
    base = wid * _CHUNK
    table_cp = pltpu.async_copy(x_hbm, table_v, sem_t)
    idx_cp = pltpu.async_copy(
        idx_hbm.at[pl.ds(base, _SUBC)], idx_v.at[pl.ds(0, _SUBC)], sem_i)
    table_cp.wait()
    out_cps = []
    for k in range(_NSUB):
        idx_cp.wait()
        if k + 1 < _NSUB:
            off_n = (k + 1) * _SUBC
            idx_cp = pltpu.async_copy(
                idx_hbm.at[pl.ds(base + off_n, _SUBC)],
                idx_v.at[pl.ds(off_n, _SUBC)], sem_i)
        off0 = k * _SUBC

        @plsc.parallel_loop(0, _SUBV, unroll=8)
        def step(i, _off0=off0):
            off = _off0 + i * _L
            iv = idx_v[pl.ds(off, _L)]
            out_v[pl.ds(off, _L)] = plsc.load_gather(table_v, [iv])

        out_cps.append(pltpu.async_copy(
            out_v.at[pl.ds(off0, _SUBC)],
            out_hbm.at[pl.ds(base + off0, _SUBC)], sem_o))
    for cp in out_cps:
        cp.wait()


_gather = pl.kernel(
    _gather_body,
    out_type=jax.ShapeDtypeStruct((_TOTAL,), jnp.float32),
    mesh=plsc.VectorSubcoreMesh(core_axis_name="c", subcore_axis_name="s"),
    scratch_types=[
        pltpu.VMEM((_TABLE,), jnp.float32),
        pltpu.VMEM((_CHUNK,), jnp.int32),
        pltpu.VMEM((_CHUNK,), jnp.float32),
        pltpu.SemaphoreType.DMA,
        pltpu.SemaphoreType.DMA,
        pltpu.SemaphoreType.DMA,
    ],
    compiler_params=pltpu.CompilerParams(needs_layout_passes=False),
)


def kernel(x, idx_tensor):
    idx_flat = idx_tensor.astype(jnp.int32).reshape(_TOTAL)
    return _gather(x, idx_flat).reshape(_SHAPE)
